# baseline (speedup 1.0000x reference)
import functools

import jax
import jax.numpy as jnp
from jax import lax
from jax.experimental import pallas as pl
from jax.experimental.pallas import tpu as pltpu
from jax.experimental.pallas import tpu_sc as plsc

B = 16384
D = 64
H1 = 256
H2 = 128

NC = 2
NS = 16
NW = NC * NS
B_PER_W = B // NW
IDX_CHUNK = 128
N_CHUNKS = B_PER_W // IDX_CHUNK


def _sc_gather_body(uid_hbm, iid_hbm, utab_hbm, itab_hbm, uout_hbm, iout_hbm,
                    uidx_v, iidx_v, urows_v, irows_v, usem, isem):
    wid = lax.axis_index("s") * NC + lax.axis_index("c")
    base = wid * B_PER_W
    pltpu.sync_copy(uid_hbm.at[pl.ds(wid * N_CHUNKS, N_CHUNKS)], uidx_v)
    pltpu.sync_copy(iid_hbm.at[pl.ds(wid * N_CHUNKS, N_CHUNKS)], iidx_v)
    copies = []
    for j in range(N_CHUNKS):
        copies.append(pltpu.async_copy(
            utab_hbm.at[uidx_v.at[j]],
            urows_v.at[pl.ds(j * IDX_CHUNK, IDX_CHUNK)], usem))
        copies.append(pltpu.async_copy(
            itab_hbm.at[iidx_v.at[j]],
            irows_v.at[pl.ds(j * IDX_CHUNK, IDX_CHUNK)], isem))
    for c in copies:
        c.wait()
    pltpu.sync_copy(urows_v, uout_hbm.at[pl.ds(base, B_PER_W)])
    pltpu.sync_copy(irows_v, iout_hbm.at[pl.ds(base, B_PER_W)])


_sc_gather = functools.partial(
    pl.kernel,
    out_type=[
        jax.ShapeDtypeStruct((B, D), jnp.float32),
        jax.ShapeDtypeStruct((B, D), jnp.float32),
    ],
    mesh=plsc.VectorSubcoreMesh(core_axis_name="c", subcore_axis_name="s"),
    scratch_types=[
        pltpu.VMEM((N_CHUNKS, IDX_CHUNK), jnp.int32),
        pltpu.VMEM((N_CHUNKS, IDX_CHUNK), jnp.int32),
        pltpu.VMEM((B_PER_W, D), jnp.float32),
        pltpu.VMEM((B_PER_W, D), jnp.float32),
        pltpu.SemaphoreType.DMA,
        pltpu.SemaphoreType.DMA,
    ],
    compiler_params=pltpu.CompilerParams(use_tc_tiling_on_sc=False),
)(_sc_gather_body)


BB = 2048


def _tc_mlp_body(ue_ref, ie_ref, w1u_ref, w1i_ref, w1d_ref, b1_ref,
                 w2_ref, b2_ref, w3_ref, b3_ref, out_ref):
    u = ue_ref[...]
    it = ie_ref[...]
    dot = jnp.sum(u * it, axis=1, keepdims=True)
    h = jnp.dot(u, w1u_ref[...], preferred_element_type=jnp.float32)
    h += jnp.dot(it, w1i_ref[...], preferred_element_type=jnp.float32)
    h += dot * w1d_ref[...][None, :] + b1_ref[...][None, :]
    h = jnp.maximum(h, 0.0)
    h2 = jnp.dot(h, w2_ref[...], preferred_element_type=jnp.float32)
    h2 = jnp.maximum(h2 + b2_ref[...][None, :], 0.0)
    p = jnp.sum(h2 * w3_ref[...][None, :], axis=1) + b3_ref[0, 0]
    out_ref[...] = p


def _tc_mlp(ue, ie, w1u, w1i, w1d, b1, w2, b2, w3, b3):
    grid = (B // BB,)
    return pl.pallas_call(
        _tc_mlp_body,
        grid=grid,
        in_specs=[
            pl.BlockSpec((BB, D), lambda i: (i, 0)),
            pl.BlockSpec((BB, D), lambda i: (i, 0)),
            pl.BlockSpec((D, H1), lambda i: (0, 0)),
            pl.BlockSpec((D, H1), lambda i: (0, 0)),
            pl.BlockSpec((H1,), lambda i: (0,)),
            pl.BlockSpec((H1,), lambda i: (0,)),
            pl.BlockSpec((H1, H2), lambda i: (0, 0)),
            pl.BlockSpec((H2,), lambda i: (0,)),
            pl.BlockSpec((H2,), lambda i: (0,)),
            pl.BlockSpec(memory_space=pltpu.SMEM),
        ],
        out_specs=pl.BlockSpec((BB,), lambda i: (i,)),
        out_shape=jax.ShapeDtypeStruct((B,), jnp.float32),
    )(ue, ie, w1u, w1i, w1d, b1, w2, b2, w3, b3)


def kernel(user_id, item_id, user_table, item_table, W1, b1, W2, b2, W3, b3):
    uid = user_id.astype(jnp.int32).reshape(B // IDX_CHUNK, IDX_CHUNK)
    iid = item_id.astype(jnp.int32).reshape(B // IDX_CHUNK, IDX_CHUNK)
    ue, ie = _sc_gather(uid, iid, user_table, item_table)
    w1t = W1.T
    w1u = w1t[:D]
    w1i = w1t[D:2 * D]
    w1d = w1t[2 * D]
    w2t = W2.T
    w3 = W3[0]
    b3s = b3.reshape(1, 1)
    return _tc_mlp(ue, ie, w1u, w1i, w1d, b1, w2t, b2, w3, b3s)


# speedup vs baseline: 1.2021x; 1.2021x over previous
import functools

import jax
import jax.numpy as jnp
from jax import lax
from jax.experimental import pallas as pl
from jax.experimental.pallas import tpu as pltpu
from jax.experimental.pallas import tpu_sc as plsc

B = 16384
D = 64
H1 = 256
H2 = 128

NC = 2
NS = 16
NW = NC * NS
B_PER_W = B // NW
IDX_CHUNK = 128
N_CHUNKS = B_PER_W // IDX_CHUNK


def _sc_gather_body(uid_hbm, iid_hbm, utab_hbm, itab_hbm, out_hbm,
                    uidx_v, iidx_v, urows_v, irows_v, usem, isem):
    wid = lax.axis_index("s") * NC + lax.axis_index("c")
    base = wid * B_PER_W
    pltpu.sync_copy(uid_hbm.at[pl.ds(wid * N_CHUNKS, N_CHUNKS)], uidx_v)
    pltpu.sync_copy(iid_hbm.at[pl.ds(wid * N_CHUNKS, N_CHUNKS)], iidx_v)
    copies = []
    for j in range(N_CHUNKS):
        r = pl.ds(j * IDX_CHUNK, IDX_CHUNK)
        copies.append(pltpu.async_copy(
            utab_hbm.at[uidx_v.at[j]], urows_v.at[r], usem))
        copies.append(pltpu.async_copy(
            itab_hbm.at[iidx_v.at[j]], irows_v.at[r], isem))
    for c in copies:
        c.wait()
    rows = pl.ds(base, B_PER_W)
    pltpu.sync_copy(urows_v, out_hbm.at[rows, pl.ds(0, D)])
    pltpu.sync_copy(irows_v, out_hbm.at[rows, pl.ds(D, D)])


_sc_gather = functools.partial(
    pl.kernel,
    out_type=jax.ShapeDtypeStruct((B, 2 * D), jnp.float32),
    mesh=plsc.VectorSubcoreMesh(core_axis_name="c", subcore_axis_name="s"),
    scratch_types=[
        pltpu.VMEM((N_CHUNKS, IDX_CHUNK), jnp.int32),
        pltpu.VMEM((N_CHUNKS, IDX_CHUNK), jnp.int32),
        pltpu.VMEM((B_PER_W, D), jnp.float32),
        pltpu.VMEM((B_PER_W, D), jnp.float32),
        pltpu.SemaphoreType.DMA,
        pltpu.SemaphoreType.DMA,
    ],
    compiler_params=pltpu.CompilerParams(use_tc_tiling_on_sc=False),
)(_sc_gather_body)


BB = 2048


def _tc_mlp_body(ui_ref, w1ui_ref, w1d_ref, b1_ref,
                 w2_ref, b2_ref, w3_ref, b3_ref, out_ref):
    ui = ui_ref[...]
    dot = jnp.sum(ui[:, :D] * ui[:, D:], axis=1, keepdims=True)
    h = jnp.dot(ui, w1ui_ref[...], preferred_element_type=jnp.float32)
    h += dot * w1d_ref[...][None, :] + b1_ref[...][None, :]
    h = jnp.maximum(h, 0.0)
    h2 = jnp.dot(h, w2_ref[...], preferred_element_type=jnp.float32)
    h2 = jnp.maximum(h2 + b2_ref[...][None, :], 0.0)
    p = jnp.sum(h2 * w3_ref[...][None, :], axis=1) + b3_ref[0]
    out_ref[...] = p


def _tc_mlp(ui, w1ui, w1d, b1, w2, b2, w3, b3):
    grid = (B // BB,)
    return pl.pallas_call(
        _tc_mlp_body,
        grid=grid,
        in_specs=[
            pl.BlockSpec((BB, 2 * D), lambda i: (i, 0)),
            pl.BlockSpec((2 * D, H1), lambda i: (0, 0)),
            pl.BlockSpec((H1,), lambda i: (0,)),
            pl.BlockSpec((H1,), lambda i: (0,)),
            pl.BlockSpec((H1, H2), lambda i: (0, 0)),
            pl.BlockSpec((H2,), lambda i: (0,)),
            pl.BlockSpec((H2,), lambda i: (0,)),
            pl.BlockSpec(memory_space=pltpu.SMEM),
        ],
        out_specs=pl.BlockSpec((BB,), lambda i: (i,)),
        out_shape=jax.ShapeDtypeStruct((B,), jnp.float32),
    )(ui, w1ui, w1d, b1, w2, b2, w3, b3)


def kernel(user_id, item_id, user_table, item_table, W1, b1, W2, b2, W3, b3):
    uid = user_id.astype(jnp.int32).reshape(B // IDX_CHUNK, IDX_CHUNK)
    iid = item_id.astype(jnp.int32).reshape(B // IDX_CHUNK, IDX_CHUNK)
    ui = _sc_gather(uid, iid, user_table, item_table)
    w1t = W1.T
    w1ui = w1t[:2 * D]
    w1d = w1t[2 * D]
    w2t = W2.T
    w3 = W3[0]
    return _tc_mlp(ui, w1ui, w1d, b1, w2t, b2, w3, b3)
